# baseline (speedup 1.0000x reference)
import functools

import jax
import jax.numpy as jnp
from jax import lax
from jax.experimental import pallas as pl
from jax.experimental.pallas import tpu as pltpu
from jax.experimental.pallas import tpu_sc as plsc

N = 10000
E = 160000
F = 256
H = 64
G = 128

BLK = 512
NPAD = 10240
NBLK = NPAD // BLK

SPLIT = (16, 64)
NC = 2
NS = 16
NW = NC * NS
CHUNK = 128
NCHT = 80
EPAD = NS * NCHT * CHUNK
RPT = NPAD // NS


def _make_agg(D, NCH0, NCH1):
    mesh = plsc.VectorSubcoreMesh(
        core_axis_name="c", subcore_axis_name="s",
        num_cores=NC, num_subcores=NS)

    assert NCH0 + NCH1 == NCHT and NCH0 % 8 == 0 and NCH1 % 8 == 0
    NSLOT = 4
    NG0 = NCH0 // NSLOT
    NG1 = NCH1 // NSLOT
    NCHMAX = max(NCH0, NCH1)

    def body(hs, zero, src2d, dst2d, out, src_all, dst_all, rows, table,
             gsem0, gsem1, ssem0, ssem1):
        c = lax.axis_index("c")
        s = lax.axis_index("s")
        r0 = s * RPT
        gsems = (gsem0, gsem1)
        ssems = (ssem0, ssem1)

        @pl.when(c == 0)
        def _():
            pltpu.sync_copy(hs.at[pl.ds(r0, RPT)], table.at[pl.ds(r0, RPT)])
            pltpu.sync_copy(src2d.at[pl.ds(s * NCH0, NCH0)],
                            src_all.at[pl.ds(0, NCH0)])
            pltpu.sync_copy(dst2d.at[pl.ds(s * NCH0, NCH0)],
                            dst_all.at[pl.ds(0, NCH0)])

        @pl.when(c != 0)
        def _():
            pltpu.sync_copy(zero.at[pl.ds(r0, RPT)], table.at[pl.ds(r0, RPT)])
            pltpu.sync_copy(src2d.at[pl.ds(NS * NCH0 + s * NCH1, NCH1)],
                            src_all.at[pl.ds(0, NCH1)])
            pltpu.sync_copy(dst2d.at[pl.ds(NS * NCH0 + s * NCH1, NCH1)],
                            dst_all.at[pl.ds(0, NCH1)])

        plsc.subcore_barrier()
        ng = jnp.where(c == 0, NG0, NG1)

        def issue_gathers(g, r):
            for b in range(NSLOT):
                pltpu.async_copy(hs.at[src_all.at[g * NSLOT + b]],
                                 rows.at[r, b], gsems[r])

        def wait_gathers(r):
            for b in range(NSLOT):
                pltpu.make_async_copy(hs.at[pl.ds(0, CHUNK)], rows.at[r, b],
                                      gsems[r]).wait()

        def issue_scatters(g, r):
            for b in range(NSLOT):
                pltpu.async_copy(rows.at[r, b],
                                 table.at[dst_all.at[g * NSLOT + b]],
                                 ssems[r], add=True)

        def wait_scatters(r):
            for b in range(NSLOT):
                pltpu.make_async_copy(rows.at[r, b], table.at[pl.ds(0, CHUNK)],
                                      ssems[r]).wait()

        issue_gathers(0, 0)

        def group_iter(i, carry):
            for r in range(2):
                g = 2 * i + r
                wait_gathers(r)
                issue_scatters(g, r)

                @pl.when(g >= 1)
                def _():
                    wait_scatters(1 - r)

                @pl.when(g + 1 < ng)
                def _():
                    issue_gathers(g + 1, 1 - r)
            return carry

        lax.fori_loop(0, ng // 2, group_iter, 0)
        wait_scatters(1)
        plsc.subcore_barrier()
        pltpu.sync_copy(table.at[pl.ds(r0, RPT)], out.at[c, pl.ds(r0, RPT)])

    return pl.kernel(
        body,
        out_type=jax.ShapeDtypeStruct((NC, NPAD, D), jnp.float32),
        mesh=mesh,
        compiler_params=pltpu.CompilerParams(use_tc_tiling_on_sc=False),
        scratch_types=[
            pltpu.VMEM((NCHMAX, CHUNK), jnp.int32),
            pltpu.VMEM((NCHMAX, CHUNK), jnp.int32),
            pltpu.VMEM((2, NSLOT, CHUNK, D), jnp.float32),
            pltpu.VMEM_SHARED((NPAD, D), jnp.float32),
            pltpu.SemaphoreType.DMA,
            pltpu.SemaphoreType.DMA,
            pltpu.SemaphoreType.DMA,
            pltpu.SemaphoreType.DMA,
        ],
    )


_make_agg = functools.lru_cache()(_make_agg)


def _agg(D, split, hs, zero, src2d, dst2d):
    return _make_agg(D, split[0], split[1])(hs, zero, src2d, dst2d)


def _matmul1_tc(x_pad, W1):
    def body(x_ref, w_ref, o_ref):
        o_ref[...] = jnp.dot(x_ref[...], w_ref[...],
                             preferred_element_type=jnp.float32)

    return pl.pallas_call(
        body,
        grid=(NBLK,),
        in_specs=[
            pl.BlockSpec((BLK, F), lambda i: (i, 0)),
            pl.BlockSpec((F, H), lambda i: (0, 0)),
        ],
        out_specs=pl.BlockSpec((BLK, H), lambda i: (i, 0)),
        out_shape=jax.ShapeDtypeStruct((NPAD, H), jnp.float32),
    )(x_pad, W1)


def _dinv_tc(p0, p1, u):
    def body(p0_ref, p1_ref, u_ref, d_ref, h_ref):
        deg = p0_ref[:, 0:1] + p1_ref[:, 0:1]
        d = jnp.where(deg > 0.0, lax.rsqrt(deg), 0.0)
        d_ref[...] = d
        h_ref[...] = u_ref[...] * d

    return pl.pallas_call(
        body,
        grid=(NBLK,),
        in_specs=[
            pl.BlockSpec((BLK, 16), lambda i: (i, 0)),
            pl.BlockSpec((BLK, 16), lambda i: (i, 0)),
            pl.BlockSpec((BLK, H), lambda i: (i, 0)),
        ],
        out_specs=[
            pl.BlockSpec((BLK, 1), lambda i: (i, 0)),
            pl.BlockSpec((BLK, H), lambda i: (i, 0)),
        ],
        out_shape=[
            jax.ShapeDtypeStruct((NPAD, 1), jnp.float32),
            jax.ShapeDtypeStruct((NPAD, H), jnp.float32),
        ],
    )(p0, p1, u)


def _mid_tc(p0, p1, dinv, b, Wn):
    def body(p0_ref, p1_ref, d_ref, b_ref, w_ref, o_ref):
        d = d_ref[...]
        t = jnp.maximum((p0_ref[...] + p1_ref[...]) * d + b_ref[...], 0.0)
        o_ref[...] = jnp.dot(t, w_ref[...], preferred_element_type=jnp.float32) * d

    return pl.pallas_call(
        body,
        grid=(NBLK,),
        in_specs=[
            pl.BlockSpec((BLK, H), lambda i: (i, 0)),
            pl.BlockSpec((BLK, H), lambda i: (i, 0)),
            pl.BlockSpec((BLK, 1), lambda i: (i, 0)),
            pl.BlockSpec((1, H), lambda i: (0, 0)),
            pl.BlockSpec((H, H), lambda i: (0, 0)),
        ],
        out_specs=pl.BlockSpec((BLK, H), lambda i: (i, 0)),
        out_shape=jax.ShapeDtypeStruct((NPAD, H), jnp.float32),
    )(p0, p1, dinv, b, Wn)


def _final_tc(p0, p1, dinv, b5, batchf, fW1, fb1, fW2, fb2):
    def body(p0_ref, p1_ref, d_ref, b_ref, bat_ref, w1_ref, c1_ref, w2_ref,
             c2_ref, o_ref, acc):
        i = pl.program_id(0)

        @pl.when(i == 0)
        def _():
            acc[...] = jnp.zeros_like(acc)

        h = jnp.maximum((p0_ref[...] + p1_ref[...]) * d_ref[...] + b_ref[...],
                        0.0)
        ids = lax.broadcasted_iota(jnp.int32, (BLK, G), 1).astype(jnp.float32)
        onehot = (bat_ref[...] == ids).astype(jnp.float32)
        acc[...] += lax.dot_general(onehot, h, (((0,), (0,)), ((), ())),
                                    preferred_element_type=jnp.float32)

        @pl.when(i == NBLK - 1)
        def _():
            g = jnp.maximum(
                jnp.dot(acc[...], w1_ref[...],
                        preferred_element_type=jnp.float32) + c1_ref[...], 0.0)
            g = jnp.dot(g, w2_ref[...],
                        preferred_element_type=jnp.float32) + c2_ref[...]
            m = jnp.max(g, axis=-1, keepdims=True)
            lse = m + jnp.log(jnp.sum(jnp.exp(g - m), axis=-1, keepdims=True))
            o_ref[...] = g - lse

    return pl.pallas_call(
        body,
        grid=(NBLK,),
        in_specs=[
            pl.BlockSpec((BLK, H), lambda i: (i, 0)),
            pl.BlockSpec((BLK, H), lambda i: (i, 0)),
            pl.BlockSpec((BLK, 1), lambda i: (i, 0)),
            pl.BlockSpec((1, H), lambda i: (0, 0)),
            pl.BlockSpec((BLK, 1), lambda i: (i, 0)),
            pl.BlockSpec((H, H), lambda i: (0, 0)),
            pl.BlockSpec((1, H), lambda i: (0, 0)),
            pl.BlockSpec((H, 1), lambda i: (0, 0)),
            pl.BlockSpec((1, 1), lambda i: (0, 0)),
        ],
        out_specs=pl.BlockSpec((G, 1), lambda i: (0, 0)),
        out_shape=jax.ShapeDtypeStruct((G, 1), jnp.float32),
        scratch_shapes=[pltpu.VMEM((G, H), jnp.float32)],
    )(p0, p1, dinv, b5, batchf, fW1, fb1, fW2, fb2)


def kernel(x, edge_index, batch, W1, b1, W2, b2, W3, b3, W4, b4, W5, b5,
           fW1, fb1, fW2, fb2):
    x_pad = jnp.pad(x, ((0, NPAD - N), (0, 0)))
    src2d = jnp.pad(edge_index[0], (0, EPAD - E),
                    constant_values=N).reshape(EPAD // CHUNK, CHUNK)
    dst2d = jnp.pad(edge_index[1], (0, EPAD - E),
                    constant_values=N).reshape(EPAD // CHUNK, CHUNK)
    batchf = jnp.pad(batch, (0, NPAD - N),
                     constant_values=G).astype(jnp.float32).reshape(NPAD, 1)

    ones16 = jnp.concatenate(
        [jnp.ones((N, 16), jnp.float32), jnp.zeros((NPAD - N, 16), jnp.float32)])
    zeros16 = jnp.zeros((NPAD, 16), jnp.float32)
    zeros64 = jnp.zeros((NPAD, H), jnp.float32)

    u = _matmul1_tc(x_pad, W1)
    degp = _agg(16, SPLIT, ones16, zeros16, src2d, dst2d)
    dinv, hs = _dinv_tc(degp[0], degp[1], u)

    weights = [(b1, W2), (b2, W3), (b3, W4), (b4, W5)]
    for bi, Wn in weights:
        p = _agg(H, SPLIT, hs, zeros64, src2d, dst2d)
        hs = _mid_tc(p[0], p[1], dinv, bi.reshape(1, H), Wn)
    p = _agg(H, SPLIT, hs, zeros64, src2d, dst2d)

    return _final_tc(p[0], p[1], dinv, b5.reshape(1, H), batchf,
                     fW1, fb1.reshape(1, H), fW2, fb2.reshape(1, 1))


# speedup vs baseline: 1.1020x; 1.1020x over previous
import functools

import jax
import jax.numpy as jnp
from jax import lax
from jax.experimental import pallas as pl
from jax.experimental.pallas import tpu as pltpu
from jax.experimental.pallas import tpu_sc as plsc

N = 10000
E = 160000
F = 256
H = 64
G = 128

BLK = 512
NPAD = 10240
NBLK = NPAD // BLK

SPLIT = (64, 16)
NC = 2
NS = 16
NW = NC * NS
CHUNK = 128
NCHT = 80
EPAD = NS * NCHT * CHUNK
RPT = NPAD // NS


def _make_agg(D, NCH0, NCH1):
    mesh = plsc.VectorSubcoreMesh(
        core_axis_name="c", subcore_axis_name="s",
        num_cores=NC, num_subcores=NS)

    assert NCH0 + NCH1 == NCHT and NCH0 % 8 == 0 and NCH1 % 8 == 0
    NSLOT = 4
    NG0 = NCH0 // NSLOT
    NG1 = NCH1 // NSLOT
    NCHMAX = max(NCH0, NCH1)

    def body(hs, zero, src2d, dst2d, out, src_all, dst_all, rows, table,
             gsem0, gsem1, ssem0, ssem1):
        c = lax.axis_index("c")
        s = lax.axis_index("s")
        r0 = s * RPT
        gsems = (gsem0, gsem1)
        ssems = (ssem0, ssem1)

        @pl.when(c == 0)
        def _():
            pltpu.sync_copy(hs.at[pl.ds(r0, RPT)], table.at[pl.ds(r0, RPT)])
            pltpu.sync_copy(src2d.at[pl.ds(s * NCH0, NCH0)],
                            src_all.at[pl.ds(0, NCH0)])
            pltpu.sync_copy(dst2d.at[pl.ds(s * NCH0, NCH0)],
                            dst_all.at[pl.ds(0, NCH0)])

        @pl.when(c != 0)
        def _():
            pltpu.sync_copy(zero.at[pl.ds(r0, RPT)], table.at[pl.ds(r0, RPT)])
            pltpu.sync_copy(src2d.at[pl.ds(NS * NCH0 + s * NCH1, NCH1)],
                            src_all.at[pl.ds(0, NCH1)])
            pltpu.sync_copy(dst2d.at[pl.ds(NS * NCH0 + s * NCH1, NCH1)],
                            dst_all.at[pl.ds(0, NCH1)])

        plsc.subcore_barrier()
        ng = jnp.where(c == 0, NG0, NG1)

        def issue_gathers(g, r):
            for b in range(NSLOT):
                pltpu.async_copy(hs.at[src_all.at[g * NSLOT + b]],
                                 rows.at[r, b], gsems[r])

        def wait_gathers(r):
            for b in range(NSLOT):
                pltpu.make_async_copy(hs.at[pl.ds(0, CHUNK)], rows.at[r, b],
                                      gsems[r]).wait()

        def issue_scatters(g, r):
            for b in range(NSLOT):
                pltpu.async_copy(rows.at[r, b],
                                 table.at[dst_all.at[g * NSLOT + b]],
                                 ssems[r], add=True)

        def wait_scatters(r):
            for b in range(NSLOT):
                pltpu.make_async_copy(rows.at[r, b], table.at[pl.ds(0, CHUNK)],
                                      ssems[r]).wait()

        issue_gathers(0, 0)

        def group_iter(i, carry):
            for r in range(2):
                g = 2 * i + r
                wait_gathers(r)
                issue_scatters(g, r)

                @pl.when(g >= 1)
                def _():
                    wait_scatters(1 - r)

                @pl.when(g + 1 < ng)
                def _():
                    issue_gathers(g + 1, 1 - r)
            return carry

        lax.fori_loop(0, ng // 2, group_iter, 0)
        wait_scatters(1)
        plsc.subcore_barrier()
        pltpu.sync_copy(table.at[pl.ds(r0, RPT)], out.at[c, pl.ds(r0, RPT)])

    return pl.kernel(
        body,
        out_type=jax.ShapeDtypeStruct((NC, NPAD, D), jnp.float32),
        mesh=mesh,
        compiler_params=pltpu.CompilerParams(use_tc_tiling_on_sc=False),
        scratch_types=[
            pltpu.VMEM((NCHMAX, CHUNK), jnp.int32),
            pltpu.VMEM((NCHMAX, CHUNK), jnp.int32),
            pltpu.VMEM((2, NSLOT, CHUNK, D), jnp.float32),
            pltpu.VMEM_SHARED((NPAD, D), jnp.float32),
            pltpu.SemaphoreType.DMA,
            pltpu.SemaphoreType.DMA,
            pltpu.SemaphoreType.DMA,
            pltpu.SemaphoreType.DMA,
        ],
    )


_make_agg = functools.lru_cache()(_make_agg)


def _agg(D, split, hs, zero, src2d, dst2d):
    return _make_agg(D, split[0], split[1])(hs, zero, src2d, dst2d)


def _matmul1_tc(x_pad, W1):
    def body(x_ref, w_ref, o_ref):
        o_ref[...] = jnp.dot(x_ref[...], w_ref[...],
                             preferred_element_type=jnp.float32)

    return pl.pallas_call(
        body,
        grid=(NBLK,),
        in_specs=[
            pl.BlockSpec((BLK, F), lambda i: (i, 0)),
            pl.BlockSpec((F, H), lambda i: (0, 0)),
        ],
        out_specs=pl.BlockSpec((BLK, H), lambda i: (i, 0)),
        out_shape=jax.ShapeDtypeStruct((NPAD, H), jnp.float32),
    )(x_pad, W1)


def _dinv_tc(p0, p1, u):
    def body(p0_ref, p1_ref, u_ref, d_ref, h_ref):
        deg = p0_ref[:, 0:1] + p1_ref[:, 0:1]
        d = jnp.where(deg > 0.0, lax.rsqrt(deg), 0.0)
        d_ref[...] = d
        h_ref[...] = u_ref[...] * d

    return pl.pallas_call(
        body,
        grid=(NBLK,),
        in_specs=[
            pl.BlockSpec((BLK, 16), lambda i: (i, 0)),
            pl.BlockSpec((BLK, 16), lambda i: (i, 0)),
            pl.BlockSpec((BLK, H), lambda i: (i, 0)),
        ],
        out_specs=[
            pl.BlockSpec((BLK, 1), lambda i: (i, 0)),
            pl.BlockSpec((BLK, H), lambda i: (i, 0)),
        ],
        out_shape=[
            jax.ShapeDtypeStruct((NPAD, 1), jnp.float32),
            jax.ShapeDtypeStruct((NPAD, H), jnp.float32),
        ],
    )(p0, p1, u)


def _mid_tc(p0, p1, dinv, b, Wn):
    def body(p0_ref, p1_ref, d_ref, b_ref, w_ref, o_ref):
        d = d_ref[...]
        t = jnp.maximum((p0_ref[...] + p1_ref[...]) * d + b_ref[...], 0.0)
        o_ref[...] = jnp.dot(t, w_ref[...], preferred_element_type=jnp.float32) * d

    return pl.pallas_call(
        body,
        grid=(NBLK,),
        in_specs=[
            pl.BlockSpec((BLK, H), lambda i: (i, 0)),
            pl.BlockSpec((BLK, H), lambda i: (i, 0)),
            pl.BlockSpec((BLK, 1), lambda i: (i, 0)),
            pl.BlockSpec((1, H), lambda i: (0, 0)),
            pl.BlockSpec((H, H), lambda i: (0, 0)),
        ],
        out_specs=pl.BlockSpec((BLK, H), lambda i: (i, 0)),
        out_shape=jax.ShapeDtypeStruct((NPAD, H), jnp.float32),
    )(p0, p1, dinv, b, Wn)


def _final_tc(p0, p1, dinv, b5, batchf, fW1, fb1, fW2, fb2):
    def body(p0_ref, p1_ref, d_ref, b_ref, bat_ref, w1_ref, c1_ref, w2_ref,
             c2_ref, o_ref, acc):
        i = pl.program_id(0)

        @pl.when(i == 0)
        def _():
            acc[...] = jnp.zeros_like(acc)

        h = jnp.maximum((p0_ref[...] + p1_ref[...]) * d_ref[...] + b_ref[...],
                        0.0)
        ids = lax.broadcasted_iota(jnp.int32, (BLK, G), 1).astype(jnp.float32)
        onehot = (bat_ref[...] == ids).astype(jnp.float32)
        acc[...] += lax.dot_general(onehot, h, (((0,), (0,)), ((), ())),
                                    preferred_element_type=jnp.float32)

        @pl.when(i == NBLK - 1)
        def _():
            g = jnp.maximum(
                jnp.dot(acc[...], w1_ref[...],
                        preferred_element_type=jnp.float32) + c1_ref[...], 0.0)
            g = jnp.dot(g, w2_ref[...],
                        preferred_element_type=jnp.float32) + c2_ref[...]
            m = jnp.max(g, axis=-1, keepdims=True)
            lse = m + jnp.log(jnp.sum(jnp.exp(g - m), axis=-1, keepdims=True))
            o_ref[...] = g - lse

    return pl.pallas_call(
        body,
        grid=(NBLK,),
        in_specs=[
            pl.BlockSpec((BLK, H), lambda i: (i, 0)),
            pl.BlockSpec((BLK, H), lambda i: (i, 0)),
            pl.BlockSpec((BLK, 1), lambda i: (i, 0)),
            pl.BlockSpec((1, H), lambda i: (0, 0)),
            pl.BlockSpec((BLK, 1), lambda i: (i, 0)),
            pl.BlockSpec((H, H), lambda i: (0, 0)),
            pl.BlockSpec((1, H), lambda i: (0, 0)),
            pl.BlockSpec((H, 1), lambda i: (0, 0)),
            pl.BlockSpec((1, 1), lambda i: (0, 0)),
        ],
        out_specs=pl.BlockSpec((G, 1), lambda i: (0, 0)),
        out_shape=jax.ShapeDtypeStruct((G, 1), jnp.float32),
        scratch_shapes=[pltpu.VMEM((G, H), jnp.float32)],
    )(p0, p1, dinv, b5, batchf, fW1, fb1, fW2, fb2)


def kernel(x, edge_index, batch, W1, b1, W2, b2, W3, b3, W4, b4, W5, b5,
           fW1, fb1, fW2, fb2):
    x_pad = jnp.pad(x, ((0, NPAD - N), (0, 0)))
    src2d = jnp.pad(edge_index[0], (0, EPAD - E),
                    constant_values=N).reshape(EPAD // CHUNK, CHUNK)
    dst2d = jnp.pad(edge_index[1], (0, EPAD - E),
                    constant_values=N).reshape(EPAD // CHUNK, CHUNK)
    batchf = jnp.pad(batch, (0, NPAD - N),
                     constant_values=G).astype(jnp.float32).reshape(NPAD, 1)

    ones16 = jnp.concatenate(
        [jnp.ones((N, 16), jnp.float32), jnp.zeros((NPAD - N, 16), jnp.float32)])
    zeros16 = jnp.zeros((NPAD, 16), jnp.float32)
    zeros64 = jnp.zeros((NPAD, H), jnp.float32)

    u = _matmul1_tc(x_pad, W1)
    degp = _agg(16, SPLIT, ones16, zeros16, src2d, dst2d)
    dinv, hs = _dinv_tc(degp[0], degp[1], u)

    weights = [(b1, W2), (b2, W3), (b3, W4), (b4, W5)]
    for bi, Wn in weights:
        p = _agg(H, SPLIT, hs, zeros64, src2d, dst2d)
        hs = _mid_tc(p[0], p[1], dinv, bi.reshape(1, H), Wn)
    p = _agg(H, SPLIT, hs, zeros64, src2d, dst2d)

    return _final_tc(p[0], p[1], dinv, b5.reshape(1, H), batchf,
                     fW1, fb1.reshape(1, H), fW2, fb2.reshape(1, 1))


# speedup vs baseline: 1.6981x; 1.5409x over previous
import functools

import jax
import jax.numpy as jnp
from jax import lax
from jax.experimental import pallas as pl
from jax.experimental.pallas import tpu as pltpu
from jax.experimental.pallas import tpu_sc as plsc

N = 10000
E = 160000
F = 256
H = 64
G = 128

BLK = 512
NPAD = 10240
NBLK = NPAD // BLK

SPLIT = (40, 40)
NC = 2
NS = 16
NW = NC * NS
CHUNK = 128
NCHT = 80
EPAD = NS * NCHT * CHUNK
RPT = NPAD // NS


def _make_agg(D, NCH0, NCH1, dt):
    mesh = plsc.VectorSubcoreMesh(
        core_axis_name="c", subcore_axis_name="s",
        num_cores=NC, num_subcores=NS)

    assert NCH0 + NCH1 == NCHT and NCH0 % 8 == 0 and NCH1 % 8 == 0
    NSLOT = 4
    NG0 = NCH0 // NSLOT
    NG1 = NCH1 // NSLOT
    NCHMAX = max(NCH0, NCH1)

    def body(hs, zero, src2d, dst2d, out0, out1, src_all, dst_all, rows, table,
             gsem0, gsem1, ssem0, ssem1):
        c = lax.axis_index("c")
        s = lax.axis_index("s")
        r0 = s * RPT
        gsems = (gsem0, gsem1)
        ssems = (ssem0, ssem1)

        @pl.when(c == 0)
        def _():
            pltpu.sync_copy(hs.at[pl.ds(r0, RPT)], table.at[pl.ds(r0, RPT)])
            pltpu.sync_copy(src2d.at[pl.ds(s * NCH0, NCH0)],
                            src_all.at[pl.ds(0, NCH0)])
            pltpu.sync_copy(dst2d.at[pl.ds(s * NCH0, NCH0)],
                            dst_all.at[pl.ds(0, NCH0)])

        @pl.when(c != 0)
        def _():
            pltpu.sync_copy(zero.at[pl.ds(r0, RPT)], table.at[pl.ds(r0, RPT)])
            pltpu.sync_copy(src2d.at[pl.ds(NS * NCH0 + s * NCH1, NCH1)],
                            src_all.at[pl.ds(0, NCH1)])
            pltpu.sync_copy(dst2d.at[pl.ds(NS * NCH0 + s * NCH1, NCH1)],
                            dst_all.at[pl.ds(0, NCH1)])

        plsc.subcore_barrier()
        ng = jnp.where(c == 0, NG0, NG1)

        def issue_gathers(g, r):
            for b in range(NSLOT):
                pltpu.async_copy(hs.at[src_all.at[g * NSLOT + b]],
                                 rows.at[r, b], gsems[r])

        def wait_gathers(r):
            for b in range(NSLOT):
                pltpu.make_async_copy(hs.at[pl.ds(0, CHUNK)], rows.at[r, b],
                                      gsems[r]).wait()

        def issue_scatters(g, r):
            for b in range(NSLOT):
                pltpu.async_copy(rows.at[r, b],
                                 table.at[dst_all.at[g * NSLOT + b]],
                                 ssems[r], add=True)

        def wait_scatters(r):
            for b in range(NSLOT):
                pltpu.make_async_copy(rows.at[r, b], table.at[pl.ds(0, CHUNK)],
                                      ssems[r]).wait()

        issue_gathers(0, 0)

        def group_iter(i, carry):
            for r in range(2):
                g = 2 * i + r
                wait_gathers(r)
                issue_scatters(g, r)

                @pl.when(g >= 1)
                def _():
                    wait_scatters(1 - r)

                @pl.when(g + 1 < ng)
                def _():
                    issue_gathers(g + 1, 1 - r)
            return carry

        lax.fori_loop(0, ng // 2, group_iter, 0)
        wait_scatters(1)
        plsc.subcore_barrier()

        @pl.when(c == 0)
        def _():
            pltpu.sync_copy(table.at[pl.ds(r0, RPT)], out0.at[pl.ds(r0, RPT)])

        @pl.when(c != 0)
        def _():
            pltpu.sync_copy(table.at[pl.ds(r0, RPT)], out1.at[pl.ds(r0, RPT)])

    return pl.kernel(
        body,
        out_type=(jax.ShapeDtypeStruct((NPAD, D), dt),
                  jax.ShapeDtypeStruct((NPAD, D), dt)),
        mesh=mesh,
        compiler_params=pltpu.CompilerParams(use_tc_tiling_on_sc=False),
        scratch_types=[
            pltpu.VMEM((NCHMAX, CHUNK), jnp.int32),
            pltpu.VMEM((NCHMAX, CHUNK), jnp.int32),
            pltpu.VMEM((2, NSLOT, CHUNK, D), dt),
            pltpu.VMEM_SHARED((NPAD, D), dt),
            pltpu.SemaphoreType.DMA,
            pltpu.SemaphoreType.DMA,
            pltpu.SemaphoreType.DMA,
            pltpu.SemaphoreType.DMA,
        ],
    )


_make_agg = functools.lru_cache()(_make_agg)


def _agg(D, split, dt, hs, zero, src2d, dst2d):
    return _make_agg(D, split[0], split[1], dt)(hs, zero, src2d, dst2d)


def _matmul1_tc(x_pad, W1):
    def body(x_ref, w_ref, o_ref):
        o_ref[...] = jnp.dot(x_ref[...], w_ref[...],
                             preferred_element_type=jnp.float32)

    return pl.pallas_call(
        body,
        grid=(NBLK,),
        in_specs=[
            pl.BlockSpec((BLK, F), lambda i: (i, 0)),
            pl.BlockSpec((F, H), lambda i: (0, 0)),
        ],
        out_specs=pl.BlockSpec((BLK, H), lambda i: (i, 0)),
        out_shape=jax.ShapeDtypeStruct((NPAD, H), jnp.float32),
    )(x_pad, W1)


def _dinv_tc(p0, p1, u):
    def body(p0_ref, p1_ref, u_ref, d_ref, h_ref):
        deg = p0_ref[:, 0:1] + p1_ref[:, 0:1]
        d = jnp.where(deg > 0.0, lax.rsqrt(deg), 0.0)
        d_ref[...] = d
        h_ref[...] = (u_ref[...] * d).astype(jnp.bfloat16)

    return pl.pallas_call(
        body,
        grid=(NBLK,),
        in_specs=[
            pl.BlockSpec((BLK, 16), lambda i: (i, 0)),
            pl.BlockSpec((BLK, 16), lambda i: (i, 0)),
            pl.BlockSpec((BLK, H), lambda i: (i, 0)),
        ],
        out_specs=[
            pl.BlockSpec((BLK, 1), lambda i: (i, 0)),
            pl.BlockSpec((BLK, H), lambda i: (i, 0)),
        ],
        out_shape=[
            jax.ShapeDtypeStruct((NPAD, 1), jnp.float32),
            jax.ShapeDtypeStruct((NPAD, H), jnp.bfloat16),
        ],
    )(p0, p1, u)


def _mid_tc(p0, p1, dinv, b, Wn):
    def body(p0_ref, p1_ref, d_ref, b_ref, w_ref, o_ref):
        d = d_ref[...]
        z = p0_ref[...].astype(jnp.float32) + p1_ref[...].astype(jnp.float32)
        t = jnp.maximum(z * d + b_ref[...], 0.0)
        o_ref[...] = (jnp.dot(t, w_ref[...], preferred_element_type=jnp.float32)
                      * d).astype(jnp.bfloat16)

    return pl.pallas_call(
        body,
        grid=(NBLK,),
        in_specs=[
            pl.BlockSpec((BLK, H), lambda i: (i, 0)),
            pl.BlockSpec((BLK, H), lambda i: (i, 0)),
            pl.BlockSpec((BLK, 1), lambda i: (i, 0)),
            pl.BlockSpec((1, H), lambda i: (0, 0)),
            pl.BlockSpec((H, H), lambda i: (0, 0)),
        ],
        out_specs=pl.BlockSpec((BLK, H), lambda i: (i, 0)),
        out_shape=jax.ShapeDtypeStruct((NPAD, H), jnp.bfloat16),
    )(p0, p1, dinv, b, Wn)


def _final_tc(p0, p1, dinv, b5, batchf, fW1, fb1, fW2, fb2):
    def body(p0_ref, p1_ref, d_ref, b_ref, bat_ref, w1_ref, c1_ref, w2_ref,
             c2_ref, o_ref, acc):
        i = pl.program_id(0)

        @pl.when(i == 0)
        def _():
            acc[...] = jnp.zeros_like(acc)

        z = p0_ref[...].astype(jnp.float32) + p1_ref[...].astype(jnp.float32)
        h = jnp.maximum(z * d_ref[...] + b_ref[...], 0.0)
        ids = lax.broadcasted_iota(jnp.int32, (BLK, G), 1).astype(jnp.float32)
        onehot = (bat_ref[...] == ids).astype(jnp.float32)
        acc[...] += lax.dot_general(onehot, h, (((0,), (0,)), ((), ())),
                                    preferred_element_type=jnp.float32)

        @pl.when(i == NBLK - 1)
        def _():
            g = jnp.maximum(
                jnp.dot(acc[...], w1_ref[...],
                        preferred_element_type=jnp.float32) + c1_ref[...], 0.0)
            g = jnp.dot(g, w2_ref[...],
                        preferred_element_type=jnp.float32) + c2_ref[...]
            m = jnp.max(g, axis=-1, keepdims=True)
            lse = m + jnp.log(jnp.sum(jnp.exp(g - m), axis=-1, keepdims=True))
            o_ref[...] = g - lse

    return pl.pallas_call(
        body,
        grid=(NBLK,),
        in_specs=[
            pl.BlockSpec((BLK, H), lambda i: (i, 0)),
            pl.BlockSpec((BLK, H), lambda i: (i, 0)),
            pl.BlockSpec((BLK, 1), lambda i: (i, 0)),
            pl.BlockSpec((1, H), lambda i: (0, 0)),
            pl.BlockSpec((BLK, 1), lambda i: (i, 0)),
            pl.BlockSpec((H, H), lambda i: (0, 0)),
            pl.BlockSpec((1, H), lambda i: (0, 0)),
            pl.BlockSpec((H, 1), lambda i: (0, 0)),
            pl.BlockSpec((1, 1), lambda i: (0, 0)),
        ],
        out_specs=pl.BlockSpec((G, 1), lambda i: (0, 0)),
        out_shape=jax.ShapeDtypeStruct((G, 1), jnp.float32),
        scratch_shapes=[pltpu.VMEM((G, H), jnp.float32)],
    )(p0, p1, dinv, b5, batchf, fW1, fb1, fW2, fb2)


def kernel(x, edge_index, batch, W1, b1, W2, b2, W3, b3, W4, b4, W5, b5,
           fW1, fb1, fW2, fb2):
    x_pad = jnp.pad(x, ((0, NPAD - N), (0, 0)))
    src2d = jnp.pad(edge_index[0], (0, EPAD - E),
                    constant_values=N).reshape(EPAD // CHUNK, CHUNK)
    dst2d = jnp.pad(edge_index[1], (0, EPAD - E),
                    constant_values=N).reshape(EPAD // CHUNK, CHUNK)
    batchf = jnp.pad(batch, (0, NPAD - N),
                     constant_values=G).astype(jnp.float32).reshape(NPAD, 1)

    ones16 = jnp.concatenate(
        [jnp.ones((N, 16), jnp.float32), jnp.zeros((NPAD - N, 16), jnp.float32)])
    zeros16 = jnp.zeros((NPAD, 16), jnp.float32)
    zeros64 = jnp.zeros((NPAD, H), jnp.bfloat16)

    u = _matmul1_tc(x_pad, W1)
    degp = _agg(16, SPLIT, jnp.float32, ones16, zeros16, src2d, dst2d)
    dinv, hs = _dinv_tc(degp[0], degp[1], u)

    weights = [(b1, W2), (b2, W3), (b3, W4), (b4, W5)]
    for bi, Wn in weights:
        p = _agg(H, SPLIT, jnp.bfloat16, hs, zeros64, src2d, dst2d)
        hs = _mid_tc(p[0], p[1], dinv, bi.reshape(1, H), Wn)
    p = _agg(H, SPLIT, jnp.bfloat16, hs, zeros64, src2d, dst2d)

    return _final_tc(p[0], p[1], dinv, b5.reshape(1, H), batchf,
                     fW1, fb1.reshape(1, H), fW2, fb2.reshape(1, 1))


# speedup vs baseline: 1.7301x; 1.0188x over previous
import functools

import jax
import jax.numpy as jnp
from jax import lax
from jax.experimental import pallas as pl
from jax.experimental.pallas import tpu as pltpu
from jax.experimental.pallas import tpu_sc as plsc

N = 10000
E = 160000
F = 256
H = 64
G = 128

BLK = 512
NPAD = 10240
NBLK = NPAD // BLK

SPLIT = (40, 40)
NC = 2
NS = 16
NW = NC * NS
CHUNK = 128
NCHT = 80
EPAD = NS * NCHT * CHUNK
RPT = NPAD // NS


def _make_agg(D, NCH0, NCH1, dt):
    mesh = plsc.VectorSubcoreMesh(
        core_axis_name="c", subcore_axis_name="s",
        num_cores=NC, num_subcores=NS)

    assert NCH0 + NCH1 == NCHT and NCH0 % 8 == 0 and NCH1 % 8 == 0
    NSLOT = 10
    NG0 = NCH0 // NSLOT
    NG1 = NCH1 // NSLOT
    NCHMAX = max(NCH0, NCH1)

    def body(hs, zero, src2d, dst2d, out0, out1, src_all, dst_all, rows, table,
             gsem0, gsem1, ssem0, ssem1):
        c = lax.axis_index("c")
        s = lax.axis_index("s")
        r0 = s * RPT
        gsems = (gsem0, gsem1)
        ssems = (ssem0, ssem1)

        @pl.when(c == 0)
        def _():
            pltpu.sync_copy(hs.at[pl.ds(r0, RPT)], table.at[pl.ds(r0, RPT)])
            pltpu.sync_copy(src2d.at[pl.ds(s * NCH0, NCH0)],
                            src_all.at[pl.ds(0, NCH0)])
            pltpu.sync_copy(dst2d.at[pl.ds(s * NCH0, NCH0)],
                            dst_all.at[pl.ds(0, NCH0)])

        @pl.when(c != 0)
        def _():
            pltpu.sync_copy(zero.at[pl.ds(r0, RPT)], table.at[pl.ds(r0, RPT)])
            pltpu.sync_copy(src2d.at[pl.ds(NS * NCH0 + s * NCH1, NCH1)],
                            src_all.at[pl.ds(0, NCH1)])
            pltpu.sync_copy(dst2d.at[pl.ds(NS * NCH0 + s * NCH1, NCH1)],
                            dst_all.at[pl.ds(0, NCH1)])

        plsc.subcore_barrier()
        ng = jnp.where(c == 0, NG0, NG1)

        def issue_gathers(g, r):
            for b in range(NSLOT):
                pltpu.async_copy(hs.at[src_all.at[g * NSLOT + b]],
                                 rows.at[r, b], gsems[r])

        def wait_gathers(r):
            for b in range(NSLOT):
                pltpu.make_async_copy(hs.at[pl.ds(0, CHUNK)], rows.at[r, b],
                                      gsems[r]).wait()

        def issue_scatters(g, r):
            for b in range(NSLOT):
                pltpu.async_copy(rows.at[r, b],
                                 table.at[dst_all.at[g * NSLOT + b]],
                                 ssems[r], add=True)

        def wait_scatters(r):
            for b in range(NSLOT):
                pltpu.make_async_copy(rows.at[r, b], table.at[pl.ds(0, CHUNK)],
                                      ssems[r]).wait()

        issue_gathers(0, 0)

        def group_iter(i, carry):
            for r in range(2):
                g = 2 * i + r
                wait_gathers(r)
                issue_scatters(g, r)

                @pl.when(g >= 1)
                def _():
                    wait_scatters(1 - r)

                @pl.when(g + 1 < ng)
                def _():
                    issue_gathers(g + 1, 1 - r)
            return carry

        lax.fori_loop(0, ng // 2, group_iter, 0)
        wait_scatters(1)
        plsc.subcore_barrier()

        @pl.when(c == 0)
        def _():
            pltpu.sync_copy(table.at[pl.ds(r0, RPT)], out0.at[pl.ds(r0, RPT)])

        @pl.when(c != 0)
        def _():
            pltpu.sync_copy(table.at[pl.ds(r0, RPT)], out1.at[pl.ds(r0, RPT)])

    return pl.kernel(
        body,
        out_type=(jax.ShapeDtypeStruct((NPAD, D), dt),
                  jax.ShapeDtypeStruct((NPAD, D), dt)),
        mesh=mesh,
        compiler_params=pltpu.CompilerParams(use_tc_tiling_on_sc=False),
        scratch_types=[
            pltpu.VMEM((NCHMAX, CHUNK), jnp.int32),
            pltpu.VMEM((NCHMAX, CHUNK), jnp.int32),
            pltpu.VMEM((2, NSLOT, CHUNK, D), dt),
            pltpu.VMEM_SHARED((NPAD, D), dt),
            pltpu.SemaphoreType.DMA,
            pltpu.SemaphoreType.DMA,
            pltpu.SemaphoreType.DMA,
            pltpu.SemaphoreType.DMA,
        ],
    )


_make_agg = functools.lru_cache()(_make_agg)


def _agg(D, split, dt, hs, zero, src2d, dst2d):
    return _make_agg(D, split[0], split[1], dt)(hs, zero, src2d, dst2d)


def _matmul1_tc(x_pad, W1):
    def body(x_ref, w_ref, o_ref):
        o_ref[...] = jnp.dot(x_ref[...], w_ref[...],
                             preferred_element_type=jnp.float32)

    return pl.pallas_call(
        body,
        grid=(NBLK,),
        in_specs=[
            pl.BlockSpec((BLK, F), lambda i: (i, 0)),
            pl.BlockSpec((F, H), lambda i: (0, 0)),
        ],
        out_specs=pl.BlockSpec((BLK, H), lambda i: (i, 0)),
        out_shape=jax.ShapeDtypeStruct((NPAD, H), jnp.float32),
    )(x_pad, W1)


def _dinv_tc(p0, p1, u):
    def body(p0_ref, p1_ref, u_ref, d_ref, h_ref):
        deg = p0_ref[:, 0:1] + p1_ref[:, 0:1]
        d = jnp.where(deg > 0.0, lax.rsqrt(deg), 0.0)
        d_ref[...] = d
        h_ref[...] = (u_ref[...] * d).astype(jnp.bfloat16)

    return pl.pallas_call(
        body,
        grid=(NBLK,),
        in_specs=[
            pl.BlockSpec((BLK, 16), lambda i: (i, 0)),
            pl.BlockSpec((BLK, 16), lambda i: (i, 0)),
            pl.BlockSpec((BLK, H), lambda i: (i, 0)),
        ],
        out_specs=[
            pl.BlockSpec((BLK, 1), lambda i: (i, 0)),
            pl.BlockSpec((BLK, H), lambda i: (i, 0)),
        ],
        out_shape=[
            jax.ShapeDtypeStruct((NPAD, 1), jnp.float32),
            jax.ShapeDtypeStruct((NPAD, H), jnp.bfloat16),
        ],
    )(p0, p1, u)


def _mid_tc(p0, p1, dinv, b, Wn):
    def body(p0_ref, p1_ref, d_ref, b_ref, w_ref, o_ref):
        d = d_ref[...]
        z = p0_ref[...].astype(jnp.float32) + p1_ref[...].astype(jnp.float32)
        t = jnp.maximum(z * d + b_ref[...], 0.0)
        o_ref[...] = (jnp.dot(t, w_ref[...], preferred_element_type=jnp.float32)
                      * d).astype(jnp.bfloat16)

    return pl.pallas_call(
        body,
        grid=(NBLK,),
        in_specs=[
            pl.BlockSpec((BLK, H), lambda i: (i, 0)),
            pl.BlockSpec((BLK, H), lambda i: (i, 0)),
            pl.BlockSpec((BLK, 1), lambda i: (i, 0)),
            pl.BlockSpec((1, H), lambda i: (0, 0)),
            pl.BlockSpec((H, H), lambda i: (0, 0)),
        ],
        out_specs=pl.BlockSpec((BLK, H), lambda i: (i, 0)),
        out_shape=jax.ShapeDtypeStruct((NPAD, H), jnp.bfloat16),
    )(p0, p1, dinv, b, Wn)


def _final_tc(p0, p1, dinv, b5, batchf, fW1, fb1, fW2, fb2):
    def body(p0_ref, p1_ref, d_ref, b_ref, bat_ref, w1_ref, c1_ref, w2_ref,
             c2_ref, o_ref, acc):
        i = pl.program_id(0)

        @pl.when(i == 0)
        def _():
            acc[...] = jnp.zeros_like(acc)

        z = p0_ref[...].astype(jnp.float32) + p1_ref[...].astype(jnp.float32)
        h = jnp.maximum(z * d_ref[...] + b_ref[...], 0.0)
        ids = lax.broadcasted_iota(jnp.int32, (BLK, G), 1).astype(jnp.float32)
        onehot = (bat_ref[...] == ids).astype(jnp.float32)
        acc[...] += lax.dot_general(onehot, h, (((0,), (0,)), ((), ())),
                                    preferred_element_type=jnp.float32)

        @pl.when(i == NBLK - 1)
        def _():
            g = jnp.maximum(
                jnp.dot(acc[...], w1_ref[...],
                        preferred_element_type=jnp.float32) + c1_ref[...], 0.0)
            g = jnp.dot(g, w2_ref[...],
                        preferred_element_type=jnp.float32) + c2_ref[...]
            m = jnp.max(g, axis=-1, keepdims=True)
            lse = m + jnp.log(jnp.sum(jnp.exp(g - m), axis=-1, keepdims=True))
            o_ref[...] = g - lse

    return pl.pallas_call(
        body,
        grid=(NBLK,),
        in_specs=[
            pl.BlockSpec((BLK, H), lambda i: (i, 0)),
            pl.BlockSpec((BLK, H), lambda i: (i, 0)),
            pl.BlockSpec((BLK, 1), lambda i: (i, 0)),
            pl.BlockSpec((1, H), lambda i: (0, 0)),
            pl.BlockSpec((BLK, 1), lambda i: (i, 0)),
            pl.BlockSpec((H, H), lambda i: (0, 0)),
            pl.BlockSpec((1, H), lambda i: (0, 0)),
            pl.BlockSpec((H, 1), lambda i: (0, 0)),
            pl.BlockSpec((1, 1), lambda i: (0, 0)),
        ],
        out_specs=pl.BlockSpec((G, 1), lambda i: (0, 0)),
        out_shape=jax.ShapeDtypeStruct((G, 1), jnp.float32),
        scratch_shapes=[pltpu.VMEM((G, H), jnp.float32)],
    )(p0, p1, dinv, b5, batchf, fW1, fb1, fW2, fb2)


def kernel(x, edge_index, batch, W1, b1, W2, b2, W3, b3, W4, b4, W5, b5,
           fW1, fb1, fW2, fb2):
    x_pad = jnp.pad(x, ((0, NPAD - N), (0, 0)))
    src2d = jnp.pad(edge_index[0], (0, EPAD - E),
                    constant_values=N).reshape(EPAD // CHUNK, CHUNK)
    dst2d = jnp.pad(edge_index[1], (0, EPAD - E),
                    constant_values=N).reshape(EPAD // CHUNK, CHUNK)
    batchf = jnp.pad(batch, (0, NPAD - N),
                     constant_values=G).astype(jnp.float32).reshape(NPAD, 1)

    ones16 = jnp.concatenate(
        [jnp.ones((N, 16), jnp.float32), jnp.zeros((NPAD - N, 16), jnp.float32)])
    zeros16 = jnp.zeros((NPAD, 16), jnp.float32)
    zeros64 = jnp.zeros((NPAD, H), jnp.bfloat16)

    u = _matmul1_tc(x_pad, W1)
    degp = _agg(16, SPLIT, jnp.float32, ones16, zeros16, src2d, dst2d)
    dinv, hs = _dinv_tc(degp[0], degp[1], u)

    weights = [(b1, W2), (b2, W3), (b3, W4), (b4, W5)]
    for bi, Wn in weights:
        p = _agg(H, SPLIT, jnp.bfloat16, hs, zeros64, src2d, dst2d)
        hs = _mid_tc(p[0], p[1], dinv, bi.reshape(1, H), Wn)
    p = _agg(H, SPLIT, jnp.bfloat16, hs, zeros64, src2d, dst2d)

    return _final_tc(p[0], p[1], dinv, b5.reshape(1, H), batchf,
                     fW1, fb1.reshape(1, H), fW2, fb2.reshape(1, 1))
